# Initial kernel scaffold; baseline (speedup 1.0000x reference)
import functools

import jax
import jax.numpy as jnp
from jax import lax
from jax.experimental import pallas as pl
from jax.experimental.pallas import tpu as pltpu
from jax.experimental.pallas import tpu_sc as plsc

_NC = 2
_NS = 16
_NW = _NC * _NS

_CHUNK = 128


def _transform_table(table, W, b2d):
    V, D = table.shape
    blk = 10000
    assert V % blk == 0

    def body(t_ref, w_ref, b_ref, o_ref):
        o_ref[...] = (
            jnp.dot(t_ref[...], w_ref[...], preferred_element_type=jnp.float32)
            + b_ref[...]
        )

    return pl.pallas_call(
        body,
        grid=(V // blk,),
        in_specs=[
            pl.BlockSpec((blk, D), lambda i: (i, 0)),
            pl.BlockSpec((D, D), lambda i: (0, 0)),
            pl.BlockSpec((1, D), lambda i: (0, 0)),
        ],
        out_specs=pl.BlockSpec((blk, D), lambda i: (i, 0)),
        out_shape=jax.ShapeDtypeStruct((V, D), jnp.float32),
    )(table, W, b2d)


def _make_gather(N, D, n_chunk):
    per_w = n_chunk * _CHUNK
    mesh = plsc.VectorSubcoreMesh(core_axis_name="c", subcore_axis_name="s")

    @functools.partial(
        pl.kernel,
        out_type=jax.ShapeDtypeStruct((N, D), jnp.float32),
        mesh=mesh,
        scratch_types=[
            pltpu.VMEM((n_chunk, _CHUNK), jnp.int32),
            pltpu.VMEM((_CHUNK, D), jnp.float32),
            pltpu.SemaphoreType.DMA,
        ],
    )
    def gather_k(idx_hbm, table2_hbm, out_hbm, idx_v, rows_v, sem):
        wid = lax.axis_index("s") * _NC + lax.axis_index("c")
        pltpu.sync_copy(idx_hbm.at[wid], idx_v)
        base = wid * per_w

        def body(j, carry):
            pltpu.async_copy(table2_hbm.at[idx_v.at[j]], rows_v, sem).wait()
            pltpu.sync_copy(rows_v, out_hbm.at[pl.ds(base + j * _CHUNK, _CHUNK)])
            return carry

        lax.fori_loop(0, n_chunk, body, 0)

    return gather_k


def kernel(token_ids, table, W, b):
    Bsz, L = token_ids.shape
    V, D = table.shape
    N = Bsz * L
    assert N % (_NW * _CHUNK) == 0
    n_chunk = N // (_NW * _CHUNK)

    table2 = _transform_table(table, W, b.reshape(1, D))
    idx = token_ids.reshape(_NW, n_chunk, _CHUNK)
    out = _make_gather(N, D, n_chunk)(idx, table2)
    return out.reshape(Bsz, L, D)


# speedup vs baseline: 3.3087x; 3.3087x over previous
import functools

import jax
import jax.numpy as jnp
from jax import lax
from jax.experimental import pallas as pl
from jax.experimental.pallas import tpu as pltpu
from jax.experimental.pallas import tpu_sc as plsc

_NC = 2
_NS = 16
_NW = _NC * _NS

_CHUNK = 128


def _transform_table(table, W, b2d):
    V, D = table.shape
    blk = 10000
    assert V % blk == 0

    def body(t_ref, w_ref, b_ref, o_ref):
        o_ref[...] = (
            jnp.dot(t_ref[...], w_ref[...], preferred_element_type=jnp.float32)
            + b_ref[...]
        )

    return pl.pallas_call(
        body,
        grid=(V // blk,),
        in_specs=[
            pl.BlockSpec((blk, D), lambda i: (i, 0)),
            pl.BlockSpec((D, D), lambda i: (0, 0)),
            pl.BlockSpec((1, D), lambda i: (0, 0)),
        ],
        out_specs=pl.BlockSpec((blk, D), lambda i: (i, 0)),
        out_shape=jax.ShapeDtypeStruct((V, D), jnp.float32),
    )(table, W, b2d)


def _make_gather(N, D, n_chunk):
    per_w = n_chunk * _CHUNK
    mesh = plsc.VectorSubcoreMesh(core_axis_name="c", subcore_axis_name="s")

    @functools.partial(
        pl.kernel,
        out_type=jax.ShapeDtypeStruct((N, D), jnp.float32),
        mesh=mesh,
        scratch_types=[
            pltpu.VMEM((n_chunk, _CHUNK), jnp.int32),
            pltpu.VMEM((_CHUNK, D), jnp.float32),
            pltpu.SemaphoreType.DMA,
        ],
        compiler_params=pltpu.CompilerParams(use_tc_tiling_on_sc=False),
    )
    def gather_k(idx_hbm, table2_hbm, out_hbm, idx_v, rows_v, sem):
        wid = lax.axis_index("s") * _NC + lax.axis_index("c")
        pltpu.sync_copy(idx_hbm.at[wid], idx_v)
        base = wid * per_w

        def body(j, carry):
            pltpu.async_copy(table2_hbm.at[idx_v.at[j]], rows_v, sem).wait()
            pltpu.sync_copy(rows_v, out_hbm.at[pl.ds(base + j * _CHUNK, _CHUNK)])
            return carry

        lax.fori_loop(0, n_chunk, body, 0)

    return gather_k


def kernel(token_ids, table, W, b):
    Bsz, L = token_ids.shape
    V, D = table.shape
    N = Bsz * L
    assert N % (_NW * _CHUNK) == 0
    n_chunk = N // (_NW * _CHUNK)

    table2 = _transform_table(table, W, b.reshape(1, D))
    idx = token_ids.reshape(_NW, n_chunk, _CHUNK)
    out = _make_gather(N, D, n_chunk)(idx, table2)
    return out.reshape(Bsz, L, D)


# speedup vs baseline: 3.6644x; 1.1075x over previous
import functools

import jax
import jax.numpy as jnp
from jax import lax
from jax.experimental import pallas as pl
from jax.experimental.pallas import tpu as pltpu
from jax.experimental.pallas import tpu_sc as plsc

_NC = 2
_NS = 16
_NW = _NC * _NS

_CHUNK = 128


def _transform_table(table, W, b2d):
    V, D = table.shape
    blk = 10000
    assert V % blk == 0

    def body(t_ref, w_ref, b_ref, o_ref):
        o_ref[...] = (
            jnp.dot(t_ref[...], w_ref[...], preferred_element_type=jnp.float32)
            + b_ref[...]
        )

    return pl.pallas_call(
        body,
        grid=(V // blk,),
        in_specs=[
            pl.BlockSpec((blk, D), lambda i: (i, 0)),
            pl.BlockSpec((D, D), lambda i: (0, 0)),
            pl.BlockSpec((1, D), lambda i: (0, 0)),
        ],
        out_specs=pl.BlockSpec((blk, D), lambda i: (i, 0)),
        out_shape=jax.ShapeDtypeStruct((V, D), jnp.float32),
    )(table, W, b2d)


def _make_gather(N, D, n_chunk, K):
    per_w = n_chunk * _CHUNK
    grp = K * _CHUNK
    n_grp = n_chunk // K
    assert n_chunk % K == 0 and n_grp % 2 == 0 and n_grp >= 4
    mesh = plsc.VectorSubcoreMesh(core_axis_name="c", subcore_axis_name="s")

    @functools.partial(
        pl.kernel,
        out_type=jax.ShapeDtypeStruct((N, D), jnp.float32),
        mesh=mesh,
        scratch_types=[
            pltpu.VMEM((n_chunk, _CHUNK), jnp.int32),
            pltpu.VMEM((2, grp, D), jnp.float32),
            pltpu.SemaphoreType.DMA,
            pltpu.SemaphoreType.DMA,
        ],
        compiler_params=pltpu.CompilerParams(use_tc_tiling_on_sc=False),
    )
    def gather_k(idx_hbm, table2_hbm, out_hbm, idx_v, rows_v, gsem, ssem):
        wid = lax.axis_index("s") * _NC + lax.axis_index("c")
        base = wid * per_w
        pltpu.sync_copy(idx_hbm.at[wid], idx_v)

        def fire_group(g, buf):
            for c in range(K):
                pltpu.async_copy(
                    table2_hbm.at[idx_v.at[g * K + c]],
                    rows_v.at[buf, pl.ds(c * _CHUNK, _CHUNK)],
                    gsem,
                )

        def drain_gathers(buf):
            pltpu.make_async_copy(
                out_hbm.at[pl.ds(0, grp)], rows_v.at[buf], gsem
            ).wait()

        def fire_store(g, buf):
            pltpu.async_copy(
                rows_v.at[buf], out_hbm.at[pl.ds(base + g * grp, grp)], ssem
            )

        def drain_store(g, buf):
            pltpu.make_async_copy(
                rows_v.at[buf], out_hbm.at[pl.ds(base + g * grp, grp)], ssem
            ).wait()

        fire_group(0, 0)
        drain_gathers(0)
        fire_store(0, 0)
        fire_group(1, 1)

        def body(k, carry):
            g = 2 * k + 1
            drain_gathers(1)
            fire_store(g, 1)
            drain_store(g - 1, 0)
            fire_group(g + 1, 0)
            drain_gathers(0)
            fire_store(g + 1, 0)
            drain_store(g, 1)
            fire_group(g + 2, 1)
            return carry

        lax.fori_loop(0, n_grp // 2 - 1, body, 0)

        drain_gathers(1)
        fire_store(n_grp - 1, 1)
        drain_store(n_grp - 2, 0)
        drain_store(n_grp - 1, 1)

    return gather_k


def kernel(token_ids, table, W, b):
    Bsz, L = token_ids.shape
    V, D = table.shape
    N = Bsz * L
    assert N % (_NW * _CHUNK) == 0
    n_chunk = N // (_NW * _CHUNK)

    table2 = _transform_table(table, W, b.reshape(1, D))
    idx = token_ids.reshape(_NW, n_chunk, _CHUNK)
    out = _make_gather(N, D, n_chunk, 5)(idx, table2)
    return out.reshape(Bsz, L, D)


# speedup vs baseline: 4.1077x; 1.1210x over previous
import functools

import jax
import jax.numpy as jnp
from jax import lax
from jax.experimental import pallas as pl
from jax.experimental.pallas import tpu as pltpu
from jax.experimental.pallas import tpu_sc as plsc

_NC = 2
_NS = 16
_NW = _NC * _NS

_WIDE = 128
_CHUNK = 80
_GB = 4


def _transform_table(tableT, W, b):
    D, V = tableT.shape
    blk = 12800
    grid = pl.cdiv(V, blk)

    def body(t_ref, w_ref, b_ref, o_ref):
        o_ref[:, :D] = (
            lax.dot_general(
                t_ref[...],
                w_ref[...],
                 dimension_numbers=(((0,), (0,)), ((), ())),
                preferred_element_type=jnp.float32,
            )
            + b_ref[...][None, :]
        )

    return pl.pallas_call(
        body,
        grid=(grid,),
        in_specs=[
            pl.BlockSpec((D, blk), lambda i: (0, i)),
            pl.BlockSpec((D, D), lambda i: (0, 0)),
            pl.BlockSpec((D,), lambda i: (0,)),
        ],
        out_specs=pl.BlockSpec((blk, _WIDE), lambda i: (i, 0)),
        out_shape=jax.ShapeDtypeStruct((V, _WIDE), jnp.float32),
    )(tableT, W, b)


def _make_gather(N, n_chunk):
    per_w = n_chunk * _CHUNK
    grp = _GB * _CHUNK
    n_grp = n_chunk // _GB
    assert n_chunk % _GB == 0 and n_grp % 2 == 0 and n_grp >= 4
    mesh = plsc.VectorSubcoreMesh(core_axis_name="c", subcore_axis_name="s")

    @functools.partial(
        pl.kernel,
        out_type=jax.ShapeDtypeStruct((N, _WIDE), jnp.float32),
        mesh=mesh,
        scratch_types=[
            pltpu.VMEM((per_w,), jnp.int32),
            pltpu.VMEM((2, grp, _WIDE), jnp.float32),
            pltpu.SemaphoreType.DMA,
            pltpu.SemaphoreType.DMA,
        ],
    )
    def gather_k(idx_hbm, table2_hbm, out_hbm, idx_v, rows_v, gsem, ssem):
        wid = lax.axis_index("s") * _NC + lax.axis_index("c")
        base = wid * per_w
        pltpu.sync_copy(idx_hbm.at[pl.ds(base, per_w)], idx_v)

        def fire_group(g, buf):
            for j in range(_GB):
                pltpu.async_copy(
                    table2_hbm.at[idx_v.at[pl.ds((g * _GB + j) * _CHUNK, _CHUNK)]],
                    rows_v.at[buf, pl.ds(j * _CHUNK, _CHUNK)],
                    gsem,
                )

        def drain_gathers(g, buf):
            for j in range(_GB):
                pltpu.make_async_copy(
                    table2_hbm.at[idx_v.at[pl.ds((g * _GB + j) * _CHUNK, _CHUNK)]],
                    rows_v.at[buf, pl.ds(j * _CHUNK, _CHUNK)],
                    gsem,
                ).wait()

        def fire_store(g, buf):
            pltpu.async_copy(
                rows_v.at[buf], out_hbm.at[pl.ds(base + g * grp, grp)], ssem
            )

        def drain_store(g, buf):
            pltpu.make_async_copy(
                rows_v.at[buf], out_hbm.at[pl.ds(base + g * grp, grp)], ssem
            ).wait()

        fire_group(0, 0)
        drain_gathers(0, 0)
        fire_store(0, 0)
        fire_group(1, 1)

        def body(k, carry):
            g = 2 * k + 1
            drain_gathers(g, 1)
            fire_store(g, 1)
            drain_store(g - 1, 0)
            fire_group(g + 1, 0)
            drain_gathers(g + 1, 0)
            fire_store(g + 1, 0)
            drain_store(g, 1)
            fire_group(g + 2, 1)
            return carry

        lax.fori_loop(0, n_grp // 2 - 1, body, 0)

        drain_gathers(n_grp - 1, 1)
        fire_store(n_grp - 1, 1)
        drain_store(n_grp - 2, 0)
        drain_store(n_grp - 1, 1)

    return gather_k


def _finalize(gath3, L, D, Bsz):
    bblk = 1024

    def body(g_ref, o_ref):
        o_ref[...] = jnp.transpose(g_ref[:, :, :D], (0, 2, 1))

    return pl.pallas_call(
        body,
        grid=(L, Bsz // bblk),
        in_specs=[pl.BlockSpec((1, bblk, _WIDE), lambda l, bb: (l, bb, 0))],
        out_specs=pl.BlockSpec((1, D, bblk), lambda l, bb: (l, 0, bb)),
        out_shape=jax.ShapeDtypeStruct((L, D, Bsz), jnp.float32),
    )(gath3)


def kernel(token_ids, table, W, b):
    Bsz, L = token_ids.shape
    V, D = table.shape
    N = Bsz * L
    assert N % (_NW * _CHUNK) == 0
    n_chunk = N // (_NW * _CHUNK)

    table2 = _transform_table(table.T, W, b)
    idx = token_ids.T.reshape(-1)
    gath = _make_gather(N, n_chunk)(idx, table2)
    outP = _finalize(gath.reshape(L, Bsz, _WIDE), L, D, Bsz)
    return jnp.transpose(outP, (2, 0, 1))
